# Initial kernel scaffold; baseline (speedup 1.0000x reference)
import jax
import jax.numpy as jnp
from jax.experimental import pallas as pl


def kernel(inputs, table):
    raise NotImplementedError("write your pallas kernel here")


# speedup vs baseline: 1.8703x; 1.8703x over previous
import functools

import jax
import jax.numpy as jnp
from jax import lax
from jax.experimental import pallas as pl
from jax.experimental.pallas import tpu as pltpu
from jax.experimental.pallas import tpu_sc as plsc

NC = 2
NS = 16
NW = NC * NS

CHUNK = 128
NBUF = 4


@functools.partial(jax.jit, static_argnames=("b_per_w", "n_chunks", "d"))
def _sc_embedding_gather(idx2d, table, *, b_per_w, n_chunks, d):
    mesh = plsc.VectorSubcoreMesh(core_axis_name="c", subcore_axis_name="s")
    total = idx2d.shape[0] * idx2d.shape[1]

    @functools.partial(
        pl.kernel,
        mesh=mesh,
        out_type=jax.ShapeDtypeStruct((total, d), jnp.float32),
        scratch_types=[
            pltpu.VMEM((n_chunks, CHUNK), jnp.int32),
            pltpu.VMEM((NBUF, CHUNK, d), jnp.float32),
            pltpu.SemaphoreType.DMA,
            pltpu.SemaphoreType.DMA,
        ],
        compiler_params=pltpu.CompilerParams(use_tc_tiling_on_sc=False),
    )
    def k(idx_hbm, table_hbm, out_hbm, idx_v, rows_v, gsem, osem):
        wid = lax.axis_index("s") * NC + lax.axis_index("c")
        base = wid * b_per_w
        pltpu.sync_copy(idx_hbm.at[pl.ds(wid * n_chunks, n_chunks)], idx_v)

        def outer(t, _):
            gathers = []
            for b in range(NBUF):
                i = t * NBUF + b
                gathers.append(
                    pltpu.async_copy(
                        table_hbm.at[idx_v.at[i]], rows_v.at[b], gsem
                    )
                )
            writes = []
            for b in range(NBUF):
                i = t * NBUF + b
                gathers[b].wait()
                writes.append(
                    pltpu.async_copy(
                        rows_v.at[b],
                        out_hbm.at[pl.ds(base + i * CHUNK, CHUNK)],
                        osem,
                    )
                )
            for w in writes:
                w.wait()
            return _

        lax.fori_loop(0, n_chunks // NBUF, outer, 0)

    return k(idx2d, table)


def kernel(inputs, table):
    b0, s = inputs.shape
    v, d = table.shape
    total = b0 * s
    b_per_w = total // NW
    n_chunks = b_per_w // CHUNK
    idx2d = inputs.reshape(NW * n_chunks, CHUNK).astype(jnp.int32)
    out = _sc_embedding_gather(
        idx2d, table, b_per_w=b_per_w, n_chunks=n_chunks, d=d
    )
    return out.reshape(b0, s, d)
